# baseline (speedup 1.0000x reference)
import jax
import jax.numpy as jnp
from jax.experimental import pallas as pl
from jax.experimental.pallas import tpu as pltpu


def _make_body(bs, batch, n_steps):
    def body(w_ref, out_ref, buf0, buf1, isem0, isem1, osem0, osem1):
        i = pl.program_id(0)
        n = pl.num_programs(0)
        bufs = [buf0, buf1]
        isems = [isem0, isem1]
        osems = [osem0, osem1]

        def in_copy(step, k):
            return pltpu.make_async_copy(
                w_ref.at[pl.ds(step * bs, bs)], bufs[k], isems[k]
            )

        def fan_copies(step, k):
            return [
                pltpu.make_async_copy(
                    bufs[k], out_ref.at[b, pl.ds(step * bs, bs)], osems[k]
                )
                for b in range(batch)
            ]

        def stage(i_const):
            k = i_const % 2
            if i_const == 0:
                in_copy(0, 0).start()
            in_copy(i_const, k).wait()
            for c in fan_copies(i_const, k):
                c.start()
            if i_const + 1 < n:
                if i_const >= 1:
                    for c in fan_copies(i_const - 1, (i_const - 1) % 2):
                        c.wait()
                in_copy(i_const + 1, (i_const + 1) % 2).start()
            else:
                if i_const >= 1:
                    for c in fan_copies(i_const - 1, (i_const - 1) % 2):
                        c.wait()
                for c in fan_copies(i_const, k):
                    c.wait()

        for step in range(n_steps):
            @pl.when(i == step)
            def _(step=step):
                stage(step)

    return body


def kernel(tokens, W_pos):
    batch, seq_len = tokens.shape
    d = W_pos.shape[1]
    bs = 2048
    grid = (seq_len // bs,)
    out = pl.pallas_call(
        _make_body(bs, batch, seq_len // bs),
        grid=grid,
        in_specs=[pl.BlockSpec(memory_space=pl.ANY)],
        out_specs=pl.BlockSpec(memory_space=pl.ANY),
        out_shape=jax.ShapeDtypeStruct((batch, seq_len, d), W_pos.dtype),
        scratch_shapes=[
            pltpu.VMEM((bs, d), W_pos.dtype),
            pltpu.VMEM((bs, d), W_pos.dtype),
            pltpu.SemaphoreType.DMA,
            pltpu.SemaphoreType.DMA,
            pltpu.SemaphoreType.DMA,
            pltpu.SemaphoreType.DMA,
        ],
    )(W_pos[:seq_len])
    return out


# speedup vs baseline: 1.0039x; 1.0039x over previous
import jax
import jax.numpy as jnp
from jax.experimental import pallas as pl
from jax.experimental.pallas import tpu as pltpu

NBUF = 3


def _make_body(bs, batch, n_steps):
    def body(w_ref, out_ref, *scratch):
        bufs = scratch[:NBUF]
        sems = scratch[NBUF:]
        i = pl.program_id(0)

        def fan_copies(step, k):
            return [
                pltpu.make_async_copy(
                    bufs[k], out_ref.at[b, pl.ds(step * bs, bs)], sems[k]
                )
                for b in range(batch)
            ]

        for step in range(n_steps):
            @pl.when(i == step)
            def _(step=step):
                k = step % NBUF
                if step >= NBUF:
                    for c in fan_copies(step - NBUF, k):
                        c.wait()
                bufs[k][...] = w_ref[...]
                for c in fan_copies(step, k):
                    c.start()
                if step == n_steps - 1:
                    for tail in range(max(0, n_steps - NBUF), n_steps):
                        for c in fan_copies(tail, tail % NBUF):
                            c.wait()

    return body


def kernel(tokens, W_pos):
    batch, seq_len = tokens.shape
    d = W_pos.shape[1]
    bs = 2048
    grid = (seq_len // bs,)
    out = pl.pallas_call(
        _make_body(bs, batch, seq_len // bs),
        grid=grid,
        in_specs=[pl.BlockSpec((bs, d), lambda i: (i, 0))],
        out_specs=pl.BlockSpec(memory_space=pl.ANY),
        out_shape=jax.ShapeDtypeStruct((batch, seq_len, d), W_pos.dtype),
        scratch_shapes=[pltpu.VMEM((bs, d), W_pos.dtype) for _ in range(NBUF)]
        + [pltpu.SemaphoreType.DMA for _ in range(NBUF)],
    )(W_pos[:seq_len])
    return out
